# Initial kernel scaffold; baseline (speedup 1.0000x reference)
import jax
import jax.numpy as jnp
from jax.experimental import pallas as pl


def kernel(user_ids, item_ids, user_embed, item_embed, W1, W2, edge_row, edge_col, edge_val):
    raise NotImplementedError("write your pallas kernel here")


# speedup vs baseline: 6.1048x; 6.1048x over previous
import functools

import jax
import jax.numpy as jnp
from jax import lax
from jax.experimental import pallas as pl
from jax.experimental.pallas import tpu as pltpu
from jax.experimental.pallas import tpu_sc as plsc

NN = 100000
D = 32
HALF = 50000
NS = 16
NC = 2
DUMMY = HALF
ACC_ROWS = 50176
ZSTRIPE = ACC_ROWS // NS
OSTRIPE = 3128
G = 128
CH = 4
ZR = 64
GROUPS = 12544
GROUPS_PER_TILE = GROUPS // NS
CHUNKS = GROUPS_PER_TILE // CH
NE_PAD = GROUPS * G
BATCH = 4096
BPW = BATCH // (NC * NS)


def _spmm_body(e_hbm, row_hbm, col_hbm, val_hbm, out_hbm,
               acc, colb, rowb, valb, idxb, rowsb, zrow, gsem):
    c = lax.axis_index("c")
    s = lax.axis_index("s")
    lo = c * HALF
    hi = lo + HALF

    zeros16 = jnp.zeros((16,), jnp.float32)

    def zb(i, carry):
        zrow[i, pl.ds(0, 16)] = zeros16
        zrow[i, pl.ds(16, 16)] = zeros16
        return carry

    lax.fori_loop(0, ZR, zb, 0)
    zbase = s * ZSTRIPE

    def zacc(k, carry):
        pltpu.sync_copy(zrow, acc.at[pl.ds(zbase + k * ZR, ZR)])
        return carry

    lax.fori_loop(0, ZSTRIPE // ZR, zacc, 0)
    plsc.subcore_barrier()

    g0 = s * GROUPS_PER_TILE

    def chunk_body(k, carry):
        gb = g0 + k * CH
        pltpu.sync_copy(row_hbm.at[pl.ds(gb, CH)], rowb)
        pltpu.sync_copy(col_hbm.at[pl.ds(gb, CH)], colb)
        pltpu.sync_copy(val_hbm.at[pl.ds(gb, CH)], valb)
        for g2 in range(CH):
            for j in range(G // 16):
                r = rowb[g2, pl.ds(j * 16, 16)]
                ok = (r >= lo) & (r < hi)
                idxb[g2, pl.ds(j * 16, 16)] = jnp.where(ok, r - lo, DUMMY)
        cps = []
        for g2 in range(CH):
            cps.append(pltpu.async_copy(e_hbm.at[colb.at[g2]],
                                        rowsb.at[g2], gsem))
        for cp in cps:
            cp.wait()
        for g2 in range(CH):
            def sc_body(j, carry2, g2=g2):
                v16 = valb[g2, pl.ds(j * 16, 16)]
                eb = j * 16
                for kk in range(16):
                    v = v16[kk]
                    rowsb[g2, eb + kk, pl.ds(0, 16)] = (
                        rowsb[g2, eb + kk, pl.ds(0, 16)] * v)
                    rowsb[g2, eb + kk, pl.ds(16, 16)] = (
                        rowsb[g2, eb + kk, pl.ds(16, 16)] * v)
                return carry2

            lax.fori_loop(0, G // 16, sc_body, 0)
        for g2 in range(CH):
            pltpu.sync_copy(rowsb.at[g2], acc.at[idxb.at[g2]], add=True)
        return carry

    lax.fori_loop(0, CHUNKS, chunk_body, 0)
    plsc.subcore_barrier()

    ob = s * OSTRIPE

    @pl.when(s < NS - 1)
    def _():
        pltpu.sync_copy(acc.at[pl.ds(ob, OSTRIPE)],
                        out_hbm.at[pl.ds(lo + ob, OSTRIPE)])

    @pl.when(s == NS - 1)
    def _():
        last = HALF - (NS - 1) * OSTRIPE
        pltpu.sync_copy(acc.at[pl.ds((NS - 1) * OSTRIPE, last)],
                        out_hbm.at[pl.ds(lo + (NS - 1) * OSTRIPE, last)])


_spmm = pl.kernel(
    _spmm_body,
    out_type=jax.ShapeDtypeStruct((NN, D), jnp.float32),
    mesh=plsc.VectorSubcoreMesh(core_axis_name="c", subcore_axis_name="s"),
    scratch_types=[
        pltpu.VMEM_SHARED((ACC_ROWS, D), jnp.float32),
        pltpu.VMEM((CH, G), jnp.int32),
        pltpu.VMEM((CH, G), jnp.int32),
        pltpu.VMEM((CH, G), jnp.float32),
        pltpu.VMEM((CH, G), jnp.int32),
        pltpu.VMEM((CH, G, D), jnp.float32),
        pltpu.VMEM((ZR, D), jnp.float32),
        pltpu.SemaphoreType.DMA,
    ],
    compiler_params=pltpu.CompilerParams(use_tc_tiling_on_sc=False),
)


def _dense_body(e_ref, g_ref, w1_ref, w2_ref, o_ref):
    e = e_ref[...]
    g = g_ref[...]
    x = jnp.dot(e + g, w1_ref[...]) + jnp.dot(g * e, w2_ref[...])
    x = jnp.where(x >= 0, x, 0.2 * x)
    n = jnp.maximum(jnp.sqrt(jnp.sum(x * x, axis=1, keepdims=True)), 1e-12)
    o_ref[...] = x / n


def _dense(E, Gc, W1Tl, W2Tl):
    BR = 4000
    return pl.pallas_call(
        _dense_body,
        grid=(NN // BR,),
        in_specs=[
            pl.BlockSpec((BR, D), lambda i: (i, 0)),
            pl.BlockSpec((BR, D), lambda i: (i, 0)),
            pl.BlockSpec((D, D), lambda i: (0, 0)),
            pl.BlockSpec((D, D), lambda i: (0, 0)),
        ],
        out_specs=pl.BlockSpec((BR, D), lambda i: (i, 0)),
        out_shape=jax.ShapeDtypeStruct((NN, D), jnp.float32),
    )(E, Gc, W1Tl, W2Tl)


def _bgather_body(e0, e1, e2, e3, uid_hbm, iid_hbm,
                  u0, u1, u2, u3, i0, i1, i2, i3,
                  uidv, iidv, rbuf, gsem):
    c = lax.axis_index("c")
    s = lax.axis_index("s")
    wid = s * NC + c
    base = wid * BPW
    pltpu.sync_copy(uid_hbm.at[pl.ds(base, BPW)], uidv)
    pltpu.sync_copy(iid_hbm.at[pl.ds(base, BPW)], iidv)
    off = jnp.full((16,), HALF, jnp.int32)
    for j in range(BPW // 16):
        iidv[pl.ds(j * 16, 16)] = iidv[pl.ds(j * 16, 16)] + off
    for tbl, out, idx in ((e0, u0, uidv), (e1, u1, uidv),
                          (e2, u2, uidv), (e3, u3, uidv),
                          (e0, i0, iidv), (e1, i1, iidv),
                          (e2, i2, iidv), (e3, i3, iidv)):
        pltpu.async_copy(tbl.at[idx], rbuf, gsem).wait()
        pltpu.sync_copy(rbuf, out.at[pl.ds(base, BPW)])


_bgather = pl.kernel(
    _bgather_body,
    out_type=tuple(jax.ShapeDtypeStruct((BATCH, D), jnp.float32)
                   for _ in range(8)),
    mesh=plsc.VectorSubcoreMesh(core_axis_name="c", subcore_axis_name="s"),
    scratch_types=[
        pltpu.VMEM((BPW,), jnp.int32),
        pltpu.VMEM((BPW,), jnp.int32),
        pltpu.VMEM((BPW, D), jnp.float32),
        pltpu.SemaphoreType.DMA,
    ],
    compiler_params=pltpu.CompilerParams(use_tc_tiling_on_sc=False),
)


def _score_body(u0, u1, u2, u3, i0, i1, i2, i3, o_ref):
    acc = jnp.dot(u0[...], i0[...].T)
    acc += jnp.dot(u1[...], i1[...].T)
    acc += jnp.dot(u2[...], i2[...].T)
    acc += jnp.dot(u3[...], i3[...].T)
    o_ref[...] = acc


def _score(us, bis):
    BU = 512
    ublk = pl.BlockSpec((BU, D), lambda i, j: (i, 0))
    iblk = pl.BlockSpec((BU, D), lambda i, j: (j, 0))
    return pl.pallas_call(
        _score_body,
        grid=(BATCH // BU, BATCH // BU),
        in_specs=[ublk] * 4 + [iblk] * 4,
        out_specs=pl.BlockSpec((BU, BU), lambda i, j: (i, j)),
        out_shape=jax.ShapeDtypeStruct((BATCH, BATCH), jnp.float32),
    )(*us, *bis)


def kernel(user_ids, item_ids, user_embed, item_embed, W1, W2,
           edge_row, edge_col, edge_val):
    ne = edge_row.shape[0]
    pad = NE_PAD - ne
    row2 = jnp.pad(edge_row.astype(jnp.int32), (0, pad)).reshape(GROUPS, G)
    col2 = jnp.pad(edge_col.astype(jnp.int32), (0, pad)).reshape(GROUPS, G)
    val2 = jnp.pad(edge_val, (0, pad)).reshape(GROUPS, G)
    W1T = jnp.swapaxes(W1, 1, 2)
    W2T = jnp.swapaxes(W2, 1, 2)

    E = jnp.concatenate([user_embed, item_embed], axis=0)
    layers = [E]
    for l in range(W1.shape[0]):
        Gc = _spmm(E, row2, col2, val2)
        E = _dense(E, Gc, W1T[l], W2T[l])
        layers.append(E)

    uid = user_ids.astype(jnp.int32)
    iid = item_ids.astype(jnp.int32)
    outs = _bgather(layers[0], layers[1], layers[2], layers[3], uid, iid)
    return _score(outs[:4], outs[4:])


# speedup vs baseline: 8.7889x; 1.4397x over previous
import jax
import jax.numpy as jnp
from jax import lax
from jax.experimental import pallas as pl
from jax.experimental.pallas import tpu as pltpu
from jax.experimental.pallas import tpu_sc as plsc

NN = 100000
D = 32
HD = 16
NS = 16
NC = 2
G = 128
CH = 4
GROUPS = 12544
GROUPS_PER_TILE = GROUPS // NS
CHUNKS = GROUPS_PER_TILE // CH
NE_PAD = GROUPS * G
ZR = 250
ZSTRIPE = NN // NS
OSTRIPE = 6256
BATCH = 4096
BPW = BATCH // (NC * NS)


def _spmm_body(el_hbm, eh_hbm, row_hbm, col_hbm, val_hbm, out_l, out_h,
               acc, colb0, colb1, rowb0, rowb1, valb0, valb1,
               rowsb0, rowsb1, zrow, sem0, sem1):
    c = lax.axis_index("c")
    s = lax.axis_index("s")

    zeros16 = jnp.zeros((16,), jnp.float32)

    def zb(i, carry):
        zrow[i] = zeros16
        return carry

    lax.fori_loop(0, ZR, zb, 0)
    zbase = s * ZSTRIPE

    def zacc(k, carry):
        pltpu.sync_copy(zrow, acc.at[pl.ds(zbase + k * ZR, ZR)])
        return carry

    lax.fori_loop(0, ZSTRIPE // ZR, zacc, 0)
    plsc.subcore_barrier()

    g0 = s * GROUPS_PER_TILE

    def load_chunk(gb, colb, rowb, valb):
        pltpu.sync_copy(row_hbm.at[pl.ds(gb, CH)], rowb)
        pltpu.sync_copy(col_hbm.at[pl.ds(gb, CH)], colb)
        pltpu.sync_copy(val_hbm.at[pl.ds(gb, CH)], valb)

    def fire(colb, rowsb, sem):
        @pl.when(c == 0)
        def _():
            for g2 in range(CH):
                pltpu.async_copy(el_hbm.at[colb.at[g2]], rowsb.at[g2], sem)

        @pl.when(c == 1)
        def _():
            for g2 in range(CH):
                pltpu.async_copy(eh_hbm.at[colb.at[g2]], rowsb.at[g2], sem)

    def drain(colb, rowsb, sem):
        for g2 in range(CH):
            pltpu.make_async_copy(el_hbm.at[colb.at[g2]],
                                  rowsb.at[g2], sem).wait()

    def scale_scatter(rowb, valb, rowsb):
        for g2 in range(CH):
            def sbody(j, carry, g2=g2):
                v16 = valb[g2, pl.ds(j * 16, 16)]
                eb = j * 16
                for kk in range(16):
                    rowsb[g2, eb + kk] = rowsb[g2, eb + kk] * v16[kk]
                return carry

            lax.fori_loop(0, G // 16, sbody, 0)
        for g2 in range(CH):
            pltpu.sync_copy(rowsb.at[g2], acc.at[rowb.at[g2]], add=True)

    load_chunk(g0, colb0, rowb0, valb0)
    fire(colb0, rowsb0, sem0)

    def body(k2, carry):
        gb1 = g0 + (2 * k2 + 1) * CH
        load_chunk(gb1, colb1, rowb1, valb1)
        fire(colb1, rowsb1, sem1)
        drain(colb0, rowsb0, sem0)
        scale_scatter(rowb0, valb0, rowsb0)

        @pl.when(k2 < CHUNKS // 2 - 1)
        def _():
            gb2 = g0 + (2 * k2 + 2) * CH
            load_chunk(gb2, colb0, rowb0, valb0)
            fire(colb0, rowsb0, sem0)

        drain(colb1, rowsb1, sem1)
        scale_scatter(rowb1, valb1, rowsb1)
        return carry

    lax.fori_loop(0, CHUNKS // 2, body, 0)
    plsc.subcore_barrier()

    ob = s * OSTRIPE

    @pl.when(s < NS - 1)
    def _():
        @pl.when(c == 0)
        def _():
            pltpu.sync_copy(acc.at[pl.ds(ob, OSTRIPE)],
                            out_l.at[pl.ds(ob, OSTRIPE)])

        @pl.when(c == 1)
        def _():
            pltpu.sync_copy(acc.at[pl.ds(ob, OSTRIPE)],
                            out_h.at[pl.ds(ob, OSTRIPE)])

    @pl.when(s == NS - 1)
    def _():
        last = NN - (NS - 1) * OSTRIPE
        lb = (NS - 1) * OSTRIPE

        @pl.when(c == 0)
        def _():
            pltpu.sync_copy(acc.at[pl.ds(lb, last)], out_l.at[pl.ds(lb, last)])

        @pl.when(c == 1)
        def _():
            pltpu.sync_copy(acc.at[pl.ds(lb, last)], out_h.at[pl.ds(lb, last)])


_spmm = pl.kernel(
    _spmm_body,
    out_type=(jax.ShapeDtypeStruct((NN, HD), jnp.float32),
              jax.ShapeDtypeStruct((NN, HD), jnp.float32)),
    mesh=plsc.VectorSubcoreMesh(core_axis_name="c", subcore_axis_name="s"),
    scratch_types=[
        pltpu.VMEM_SHARED((NN, HD), jnp.float32),
        pltpu.VMEM((CH, G), jnp.int32),
        pltpu.VMEM((CH, G), jnp.int32),
        pltpu.VMEM((CH, G), jnp.int32),
        pltpu.VMEM((CH, G), jnp.int32),
        pltpu.VMEM((CH, G), jnp.float32),
        pltpu.VMEM((CH, G), jnp.float32),
        pltpu.VMEM((CH, G, HD), jnp.float32),
        pltpu.VMEM((CH, G, HD), jnp.float32),
        pltpu.VMEM((ZR, HD), jnp.float32),
        pltpu.SemaphoreType.DMA,
        pltpu.SemaphoreType.DMA,
    ],
    compiler_params=pltpu.CompilerParams(use_tc_tiling_on_sc=False),
)


def _dense_body(el_ref, eh_ref, gl_ref, gh_ref, w1_ref, w2_ref,
                ol_ref, oh_ref):
    e = jnp.concatenate([el_ref[...], eh_ref[...]], axis=1)
    g = jnp.concatenate([gl_ref[...], gh_ref[...]], axis=1)
    x = jnp.dot(e + g, w1_ref[...]) + jnp.dot(g * e, w2_ref[...])
    x = jnp.where(x >= 0, x, 0.2 * x)
    n = jnp.maximum(jnp.sqrt(jnp.sum(x * x, axis=1, keepdims=True)), 1e-12)
    x = x / n
    ol_ref[...] = x[:, :HD]
    oh_ref[...] = x[:, HD:]


def _dense(el, eh, gl, gh, W1Tl, W2Tl):
    BR = 4000
    half = pl.BlockSpec((BR, HD), lambda i: (i, 0))
    wspec = pl.BlockSpec((D, D), lambda i: (0, 0))
    hs = jax.ShapeDtypeStruct((NN, HD), jnp.float32)
    return pl.pallas_call(
        _dense_body,
        grid=(NN // BR,),
        in_specs=[half, half, half, half, wspec, wspec],
        out_specs=(half, half),
        out_shape=(hs, hs),
    )(el, eh, gl, gh, W1Tl, W2Tl)


def _bgather_body(t0, t1, t2, t3, t4, t5, t6, t7, uid_hbm, iid_hbm,
                  u_out, i_out, uidv, iidv, rbuf, gsem):
    c = lax.axis_index("c")
    s = lax.axis_index("s")
    wid = s * NC + c
    base = wid * BPW
    pltpu.sync_copy(uid_hbm.at[pl.ds(base, BPW)], uidv)
    pltpu.sync_copy(iid_hbm.at[pl.ds(base, BPW)], iidv)
    off = jnp.full((16,), NN // 2, jnp.int32)
    for j in range(BPW // 16):
        iidv[pl.ds(j * 16, 16)] = iidv[pl.ds(j * 16, 16)] + off
    tbls = (t0, t1, t2, t3, t4, t5, t6, t7)
    for p in range(8):
        pltpu.async_copy(tbls[p].at[uidv], rbuf, gsem).wait()
        pltpu.sync_copy(rbuf, u_out.at[p, pl.ds(base, BPW)])
    for p in range(8):
        pltpu.async_copy(tbls[p].at[iidv], rbuf, gsem).wait()
        pltpu.sync_copy(rbuf, i_out.at[p, pl.ds(base, BPW)])


_bgather = pl.kernel(
    _bgather_body,
    out_type=(jax.ShapeDtypeStruct((8, BATCH, HD), jnp.float32),
              jax.ShapeDtypeStruct((8, BATCH, HD), jnp.float32)),
    mesh=plsc.VectorSubcoreMesh(core_axis_name="c", subcore_axis_name="s"),
    scratch_types=[
        pltpu.VMEM((BPW,), jnp.int32),
        pltpu.VMEM((BPW,), jnp.int32),
        pltpu.VMEM((BPW, HD), jnp.float32),
        pltpu.SemaphoreType.DMA,
    ],
    compiler_params=pltpu.CompilerParams(use_tc_tiling_on_sc=False),
)


def _score_body(u_ref, i_ref, o_ref):
    acc = jnp.dot(u_ref[0], i_ref[0].T)
    for p in range(1, 8):
        acc += jnp.dot(u_ref[p], i_ref[p].T)
    o_ref[...] = acc


def _score(u_stack, i_stack):
    BU = 512
    return pl.pallas_call(
        _score_body,
        grid=(BATCH // BU, BATCH // BU),
        in_specs=[pl.BlockSpec((8, BU, HD), lambda i, j: (0, i, 0)),
                  pl.BlockSpec((8, BU, HD), lambda i, j: (0, j, 0))],
        out_specs=pl.BlockSpec((BU, BU), lambda i, j: (i, j)),
        out_shape=jax.ShapeDtypeStruct((BATCH, BATCH), jnp.float32),
    )(u_stack, i_stack)


def kernel(user_ids, item_ids, user_embed, item_embed, W1, W2,
           edge_row, edge_col, edge_val):
    ne = edge_row.shape[0]
    pad = NE_PAD - ne
    row2 = jnp.pad(edge_row.astype(jnp.int32), (0, pad)).reshape(GROUPS, G)
    col2 = jnp.pad(edge_col.astype(jnp.int32), (0, pad)).reshape(GROUPS, G)
    val2 = jnp.pad(edge_val, (0, pad)).reshape(GROUPS, G)
    W1T = jnp.swapaxes(W1, 1, 2)
    W2T = jnp.swapaxes(W2, 1, 2)

    el = jnp.concatenate([user_embed[:, :HD], item_embed[:, :HD]], axis=0)
    eh = jnp.concatenate([user_embed[:, HD:], item_embed[:, HD:]], axis=0)
    halves = [(el, eh)]
    for l in range(W1.shape[0]):
        gl, gh = _spmm(el, eh, row2, col2, val2)
        el, eh = _dense(el, eh, gl, gh, W1T[l], W2T[l])
        halves.append((el, eh))

    uid = user_ids.astype(jnp.int32)
    iid = item_ids.astype(jnp.int32)
    tbls = [h for pair in halves for h in pair]
    u_stack, i_stack = _bgather(*tbls, uid, iid)
    return _score(u_stack, i_stack)


# speedup vs baseline: 14.1309x; 1.6078x over previous
import jax
import jax.numpy as jnp
from jax import lax
from jax.experimental import pallas as pl
from jax.experimental.pallas import tpu as pltpu
from jax.experimental.pallas import tpu_sc as plsc

NN = 100000
D = 32
HD = 16
NS = 16
NC = 2
G = 128
CH = 4
GROUPS = 12544
GROUPS_PER_TILE = GROUPS // NS
CHUNKS = GROUPS_PER_TILE // CH
NE_PAD = GROUPS * G
ZR = 250
ZSTRIPE = NN // NS
OSTRIPE = 6256
BATCH = 4096
BPW = BATCH // (NC * NS)


def _spmm_body(el_hbm, eh_hbm, row_hbm, col_hbm, val_hbm, out_l, out_h,
               acc, colb0, colb1, rowb0, rowb1, valb0, valb1,
               idxb0, idxb1, rowsb0, rowsb1, zrow,
               sem_l0, sem_l1, sem_g0, sem_g1, sem_s0, sem_s1, sem_z):
    c = lax.axis_index("c")
    s = lax.axis_index("s")
    g0 = s * GROUPS_PER_TILE

    def load_chunk(gb, colb, rowb, valb, sem):
        pltpu.async_copy(row_hbm.at[pl.ds(gb, CH)], rowb, sem)
        pltpu.async_copy(col_hbm.at[pl.ds(gb, CH)], colb, sem)
        pltpu.async_copy(val_hbm.at[pl.ds(gb, CH)], valb, sem)

    def wait_chunk(colb, rowb, valb, sem):
        pltpu.make_async_copy(row_hbm.at[pl.ds(0, CH)], rowb, sem).wait()
        pltpu.make_async_copy(col_hbm.at[pl.ds(0, CH)], colb, sem).wait()
        pltpu.make_async_copy(val_hbm.at[pl.ds(0, CH)], valb, sem).wait()

    def fire(colb, rowsb, sem):
        @pl.when(c == 0)
        def _():
            for g2 in range(CH):
                pltpu.async_copy(el_hbm.at[colb.at[g2]], rowsb.at[g2], sem)

        @pl.when(c == 1)
        def _():
            for g2 in range(CH):
                pltpu.async_copy(eh_hbm.at[colb.at[g2]], rowsb.at[g2], sem)

    def drain(colb, rowsb, sem):
        for g2 in range(CH):
            pltpu.make_async_copy(el_hbm.at[colb.at[g2]],
                                  rowsb.at[g2], sem).wait()

    def scale(valb, rowsb):
        for g2 in range(CH):
            def sbody(j, carry, g2=g2):
                v16 = valb[g2, pl.ds(j * 16, 16)]
                eb = j * 16
                for kk in range(16):
                    rowsb[g2, eb + kk] = rowsb[g2, eb + kk] * v16[kk]
                return carry

            lax.fori_loop(0, G // 16, sbody, 0)

    def copy_idx(rowb, idxb):
        for g2 in range(CH):
            def cbody(j, carry, g2=g2):
                idxb[g2, pl.ds(j * 16, 16)] = rowb[g2, pl.ds(j * 16, 16)]
                return carry

            lax.fori_loop(0, G // 16, cbody, 0)

    def fire_scatter(rowsb, idxb, sem):
        for g2 in range(CH):
            pltpu.async_copy(rowsb.at[g2], acc.at[idxb.at[g2]], sem, add=True)

    def wait_scatter(rowsb, idxb, sem):
        for g2 in range(CH):
            pltpu.make_async_copy(rowsb.at[g2], acc.at[idxb.at[g2]],
                                  sem).wait()

    load_chunk(g0, colb0, rowb0, valb0, sem_l0)
    load_chunk(g0 + CH, colb1, rowb1, valb1, sem_l1)

    zeros16 = jnp.zeros((16,), jnp.float32)

    def zb(i, carry):
        zrow[i] = zeros16
        return carry

    lax.fori_loop(0, ZR, zb, 0)
    zbase = s * ZSTRIPE
    for k in range(ZSTRIPE // ZR):
        pltpu.async_copy(zrow, acc.at[pl.ds(zbase + k * ZR, ZR)], sem_z)
    wait_chunk(colb0, rowb0, valb0, sem_l0)
    for k in range(ZSTRIPE // ZR):
        pltpu.make_async_copy(zrow, acc.at[pl.ds(zbase, ZR)], sem_z).wait()
    plsc.subcore_barrier()
    fire(colb0, rowsb0, sem_g0)

    def body(k2, carry):
        a = 2 * k2
        b = a + 1

        @pl.when(k2 > 0)
        def _():
            wait_scatter(rowsb1, idxb1, sem_s1)

        wait_chunk(colb1, rowb1, valb1, sem_l1)
        fire(colb1, rowsb1, sem_g1)
        drain(colb0, rowsb0, sem_g0)
        scale(valb0, rowsb0)
        copy_idx(rowb0, idxb0)
        fire_scatter(rowsb0, idxb0, sem_s0)

        @pl.when(k2 < CHUNKS // 2 - 1)
        def _():
            load_chunk(g0 + (a + 2) * CH, colb0, rowb0, valb0, sem_l0)

        drain(colb1, rowsb1, sem_g1)
        scale(valb1, rowsb1)
        copy_idx(rowb1, idxb1)
        wait_scatter(rowsb0, idxb0, sem_s0)
        fire_scatter(rowsb1, idxb1, sem_s1)

        @pl.when(k2 < CHUNKS // 2 - 1)
        def _():
            wait_chunk(colb0, rowb0, valb0, sem_l0)
            fire(colb0, rowsb0, sem_g0)
            load_chunk(g0 + (b + 2) * CH, colb1, rowb1, valb1, sem_l1)

        return carry

    lax.fori_loop(0, CHUNKS // 2, body, 0)
    wait_scatter(rowsb1, idxb1, sem_s1)
    plsc.subcore_barrier()

    ob = s * OSTRIPE

    @pl.when(s < NS - 1)
    def _():
        @pl.when(c == 0)
        def _():
            pltpu.sync_copy(acc.at[pl.ds(ob, OSTRIPE)],
                            out_l.at[pl.ds(ob, OSTRIPE)])

        @pl.when(c == 1)
        def _():
            pltpu.sync_copy(acc.at[pl.ds(ob, OSTRIPE)],
                            out_h.at[pl.ds(ob, OSTRIPE)])

    @pl.when(s == NS - 1)
    def _():
        last = NN - (NS - 1) * OSTRIPE
        lb = (NS - 1) * OSTRIPE

        @pl.when(c == 0)
        def _():
            pltpu.sync_copy(acc.at[pl.ds(lb, last)], out_l.at[pl.ds(lb, last)])

        @pl.when(c == 1)
        def _():
            pltpu.sync_copy(acc.at[pl.ds(lb, last)], out_h.at[pl.ds(lb, last)])


_spmm = pl.kernel(
    _spmm_body,
    out_type=(jax.ShapeDtypeStruct((NN, HD), jnp.float32),
              jax.ShapeDtypeStruct((NN, HD), jnp.float32)),
    mesh=plsc.VectorSubcoreMesh(core_axis_name="c", subcore_axis_name="s"),
    scratch_types=[
        pltpu.VMEM_SHARED((NN, HD), jnp.float32),
        pltpu.VMEM((CH, G), jnp.int32),
        pltpu.VMEM((CH, G), jnp.int32),
        pltpu.VMEM((CH, G), jnp.int32),
        pltpu.VMEM((CH, G), jnp.int32),
        pltpu.VMEM((CH, G), jnp.float32),
        pltpu.VMEM((CH, G), jnp.float32),
        pltpu.VMEM((CH, G), jnp.int32),
        pltpu.VMEM((CH, G), jnp.int32),
        pltpu.VMEM((CH, G, HD), jnp.float32),
        pltpu.VMEM((CH, G, HD), jnp.float32),
        pltpu.VMEM((ZR, HD), jnp.float32),
        pltpu.SemaphoreType.DMA,
        pltpu.SemaphoreType.DMA,
        pltpu.SemaphoreType.DMA,
        pltpu.SemaphoreType.DMA,
        pltpu.SemaphoreType.DMA,
        pltpu.SemaphoreType.DMA,
        pltpu.SemaphoreType.DMA,
    ],
    compiler_params=pltpu.CompilerParams(use_tc_tiling_on_sc=False),
)


def _dense_math(e, g, w1, w2):
    x = jnp.dot(e + g, w1) + jnp.dot(g * e, w2)
    x = jnp.where(x >= 0, x, 0.2 * x)
    n = jnp.maximum(jnp.sqrt(jnp.sum(x * x, axis=1, keepdims=True)), 1e-12)
    return x / n


def _dense_body(el_ref, eh_ref, gl_ref, gh_ref, w1_ref, w2_ref,
                ol_ref, oh_ref):
    e = jnp.concatenate([el_ref[...], eh_ref[...]], axis=1)
    g = jnp.concatenate([gl_ref[...], gh_ref[...]], axis=1)
    x = _dense_math(e, g, w1_ref[...], w2_ref[...])
    ol_ref[...] = x[:, :HD]
    oh_ref[...] = x[:, HD:]


def _dense(el, eh, gl, gh, W1Tl, W2Tl):
    BR = 4000
    half = pl.BlockSpec((BR, HD), lambda i: (i, 0))
    wspec = pl.BlockSpec((D, D), lambda i: (0, 0))
    hs = jax.ShapeDtypeStruct((NN, HD), jnp.float32)
    return pl.pallas_call(
        _dense_body,
        grid=(NN // BR,),
        in_specs=[half, half, half, half, wspec, wspec],
        out_specs=(half, half),
        out_shape=(hs, hs),
    )(el, eh, gl, gh, W1Tl, W2Tl)


def _bgather_body(t0, t1, t2, t3, t4, t5, t6, t7, uid_hbm, iid_hbm,
                  u_out, i_out, uidv, iidv, rbuf, gsem):
    c = lax.axis_index("c")
    s = lax.axis_index("s")
    wid = s * NC + c
    base = wid * BPW
    pltpu.sync_copy(uid_hbm.at[pl.ds(base, BPW)], uidv)
    pltpu.sync_copy(iid_hbm.at[pl.ds(base, BPW)], iidv)
    off = jnp.full((16,), NN // 2, jnp.int32)
    for j in range(BPW // 16):
        iidv[pl.ds(j * 16, 16)] = iidv[pl.ds(j * 16, 16)] + off
    tbls = (t0, t1, t2, t3, t4, t5, t6, t7)
    for p in range(8):
        pltpu.async_copy(tbls[p].at[uidv], rbuf.at[p], gsem)
    for p in range(8):
        pltpu.make_async_copy(tbls[p].at[uidv], rbuf.at[p], gsem).wait()
        pltpu.sync_copy(rbuf.at[p], u_out.at[p, pl.ds(base, BPW)])
    for p in range(8):
        pltpu.async_copy(tbls[p].at[iidv], rbuf.at[p], gsem)
    for p in range(8):
        pltpu.make_async_copy(tbls[p].at[iidv], rbuf.at[p], gsem).wait()
        pltpu.sync_copy(rbuf.at[p], i_out.at[p, pl.ds(base, BPW)])


_bgather = pl.kernel(
    _bgather_body,
    out_type=(jax.ShapeDtypeStruct((8, BATCH, HD), jnp.float32),
              jax.ShapeDtypeStruct((8, BATCH, HD), jnp.float32)),
    mesh=plsc.VectorSubcoreMesh(core_axis_name="c", subcore_axis_name="s"),
    scratch_types=[
        pltpu.VMEM((BPW,), jnp.int32),
        pltpu.VMEM((BPW,), jnp.int32),
        pltpu.VMEM((8, BPW, HD), jnp.float32),
        pltpu.SemaphoreType.DMA,
    ],
    compiler_params=pltpu.CompilerParams(use_tc_tiling_on_sc=False),
)


def _score_body(u_ref, i_ref, w1_ref, w2_ref, o_ref):
    w1 = w1_ref[...]
    w2 = w2_ref[...]
    ue = jnp.concatenate([u_ref[4], u_ref[5]], axis=1)
    ug = jnp.concatenate([u_ref[6], u_ref[7]], axis=1)
    u3 = _dense_math(ue, ug, w1, w2)
    ie = jnp.concatenate([i_ref[4], i_ref[5]], axis=1)
    ig = jnp.concatenate([i_ref[6], i_ref[7]], axis=1)
    i3 = _dense_math(ie, ig, w1, w2)
    acc = jnp.dot(u3, i3.T)
    for p in range(6):
        acc += jnp.dot(u_ref[p], i_ref[p].T)
    o_ref[...] = acc


def _score(u_stack, i_stack, W1Tl, W2Tl):
    BU = 512
    wspec = pl.BlockSpec((D, D), lambda i, j: (0, 0))
    return pl.pallas_call(
        _score_body,
        grid=(BATCH // BU, BATCH // BU),
        in_specs=[pl.BlockSpec((8, BU, HD), lambda i, j: (0, i, 0)),
                  pl.BlockSpec((8, BU, HD), lambda i, j: (0, j, 0)),
                  wspec, wspec],
        out_specs=pl.BlockSpec((BU, BU), lambda i, j: (i, j)),
        out_shape=jax.ShapeDtypeStruct((BATCH, BATCH), jnp.float32),
    )(u_stack, i_stack, W1Tl, W2Tl)


def kernel(user_ids, item_ids, user_embed, item_embed, W1, W2,
           edge_row, edge_col, edge_val):
    ne = edge_row.shape[0]
    pad = NE_PAD - ne
    row2 = jnp.pad(edge_row.astype(jnp.int32), (0, pad)).reshape(GROUPS, G)
    col2 = jnp.pad(edge_col.astype(jnp.int32), (0, pad)).reshape(GROUPS, G)
    val2 = jnp.pad(edge_val, (0, pad)).reshape(GROUPS, G)
    W1T = jnp.swapaxes(W1, 1, 2)
    W2T = jnp.swapaxes(W2, 1, 2)

    el = jnp.concatenate([user_embed[:, :HD], item_embed[:, :HD]], axis=0)
    eh = jnp.concatenate([user_embed[:, HD:], item_embed[:, HD:]], axis=0)
    tbls = [el, eh]
    n_layers = W1.shape[0]
    for l in range(n_layers - 1):
        gl, gh = _spmm(el, eh, row2, col2, val2)
        el, eh = _dense(el, eh, gl, gh, W1T[l], W2T[l])
        tbls += [el, eh]
    g3l, g3h = _spmm(el, eh, row2, col2, val2)
    tbls += [g3l, g3h]

    uid = user_ids.astype(jnp.int32)
    iid = item_ids.astype(jnp.int32)
    u_stack, i_stack = _bgather(*tbls, uid, iid)
    return _score(u_stack, i_stack, W1T[n_layers - 1], W2T[n_layers - 1])


# speedup vs baseline: 19.6650x; 1.3916x over previous
import jax
import jax.numpy as jnp
from jax import lax
from jax.experimental import pallas as pl
from jax.experimental.pallas import tpu as pltpu
from jax.experimental.pallas import tpu_sc as plsc

NN = 100000
NNP = 100352
D = 32
HD = 16
NS = 16
NC = 2
G = 128
CH = 4
GROUPS = 12544
GROUPS_PER_TILE = GROUPS // NS
CHUNKS = GROUPS_PER_TILE // CH
NE_PAD = GROUPS * G
ZR = 128
ZSTRIPE = NNP // NS
ROWS128 = NNP * HD // 128
BR = ROWS128 // 8
BATCH = 4096
BPW = BATCH // (NC * NS)


def _spmm_body(el_hbm, eh_hbm, row_hbm, col_hbm, val_hbm, out_l, out_h,
               acc, colb0, colb1, rowb0, rowb1, valb0, valb1,
               idxb0, idxb1, rowsb0, rowsb1, zrow,
               sem_l0, sem_l1, sem_g0, sem_g1, sem_s0, sem_s1, sem_z):
    c = lax.axis_index("c")
    s = lax.axis_index("s")
    g0 = s * GROUPS_PER_TILE

    def load_chunk(gb, colb, rowb, valb, sem):
        pltpu.async_copy(row_hbm.at[pl.ds(gb, CH)], rowb, sem)
        pltpu.async_copy(col_hbm.at[pl.ds(gb, CH)], colb, sem)
        pltpu.async_copy(val_hbm.at[pl.ds(gb, CH)], valb, sem)

    def wait_chunk(colb, rowb, valb, sem):
        pltpu.make_async_copy(row_hbm.at[pl.ds(0, CH)], rowb, sem).wait()
        pltpu.make_async_copy(col_hbm.at[pl.ds(0, CH)], colb, sem).wait()
        pltpu.make_async_copy(val_hbm.at[pl.ds(0, CH)], valb, sem).wait()

    def fire(colb, rowsb, sem):
        @pl.when(c == 0)
        def _():
            for g2 in range(CH):
                pltpu.async_copy(el_hbm.at[colb.at[g2]], rowsb.at[g2], sem)

        @pl.when(c == 1)
        def _():
            for g2 in range(CH):
                pltpu.async_copy(eh_hbm.at[colb.at[g2]], rowsb.at[g2], sem)

    def drain(colb, rowsb, sem):
        for g2 in range(CH):
            pltpu.make_async_copy(el_hbm.at[colb.at[g2]],
                                  rowsb.at[g2], sem).wait()

    def scale(valb, rowsb):
        for g2 in range(CH):
            def sbody(j, carry, g2=g2):
                v16 = valb[g2, pl.ds(j * 16, 16)]
                eb = j * 16
                for kk in range(16):
                    rowsb[g2, eb + kk] = rowsb[g2, eb + kk] * v16[kk]
                return carry

            lax.fori_loop(0, G // 16, sbody, 0)

    def copy_idx(rowb, idxb):
        for g2 in range(CH):
            def cbody(j, carry, g2=g2):
                idxb[g2, pl.ds(j * 16, 16)] = rowb[g2, pl.ds(j * 16, 16)]
                return carry

            lax.fori_loop(0, G // 16, cbody, 0)

    def fire_scatter(rowsb, idxb, sem):
        for g2 in range(CH):
            pltpu.async_copy(rowsb.at[g2], acc.at[idxb.at[g2]], sem, add=True)

    def wait_scatter(rowsb, idxb, sem):
        for g2 in range(CH):
            pltpu.make_async_copy(rowsb.at[g2], acc.at[idxb.at[g2]],
                                  sem).wait()

    load_chunk(g0, colb0, rowb0, valb0, sem_l0)
    load_chunk(g0 + CH, colb1, rowb1, valb1, sem_l1)

    zeros16 = jnp.zeros((16,), jnp.float32)

    def zb(i, carry):
        zrow[i] = zeros16
        return carry

    lax.fori_loop(0, ZR, zb, 0)
    zbase = s * ZSTRIPE
    for k in range(ZSTRIPE // ZR):
        pltpu.async_copy(zrow, acc.at[pl.ds(zbase + k * ZR, ZR)], sem_z)
    wait_chunk(colb0, rowb0, valb0, sem_l0)
    for k in range(ZSTRIPE // ZR):
        pltpu.make_async_copy(zrow, acc.at[pl.ds(zbase, ZR)], sem_z).wait()
    plsc.subcore_barrier()
    fire(colb0, rowsb0, sem_g0)

    def body(k2, carry):
        a = 2 * k2
        b = a + 1

        @pl.when(k2 > 0)
        def _():
            wait_scatter(rowsb1, idxb1, sem_s1)

        wait_chunk(colb1, rowb1, valb1, sem_l1)
        fire(colb1, rowsb1, sem_g1)
        drain(colb0, rowsb0, sem_g0)
        scale(valb0, rowsb0)
        copy_idx(rowb0, idxb0)
        fire_scatter(rowsb0, idxb0, sem_s0)

        @pl.when(k2 < CHUNKS // 2 - 1)
        def _():
            load_chunk(g0 + (a + 2) * CH, colb0, rowb0, valb0, sem_l0)

        drain(colb1, rowsb1, sem_g1)
        scale(valb1, rowsb1)
        copy_idx(rowb1, idxb1)
        wait_scatter(rowsb0, idxb0, sem_s0)
        fire_scatter(rowsb1, idxb1, sem_s1)

        @pl.when(k2 < CHUNKS // 2 - 1)
        def _():
            wait_chunk(colb0, rowb0, valb0, sem_l0)
            fire(colb0, rowsb0, sem_g0)
            load_chunk(g0 + (b + 2) * CH, colb1, rowb1, valb1, sem_l1)

        return carry

    lax.fori_loop(0, CHUNKS // 2, body, 0)
    wait_scatter(rowsb1, idxb1, sem_s1)
    plsc.subcore_barrier()

    ob = s * ZSTRIPE

    @pl.when(c == 0)
    def _():
        pltpu.sync_copy(acc.at[pl.ds(ob, ZSTRIPE)],
                        out_l.at[pl.ds(ob, ZSTRIPE)])

    @pl.when(c == 1)
    def _():
        pltpu.sync_copy(acc.at[pl.ds(ob, ZSTRIPE)],
                        out_h.at[pl.ds(ob, ZSTRIPE)])


_spmm = pl.kernel(
    _spmm_body,
    out_type=(jax.ShapeDtypeStruct((NNP, HD), jnp.float32),
              jax.ShapeDtypeStruct((NNP, HD), jnp.float32)),
    mesh=plsc.VectorSubcoreMesh(core_axis_name="c", subcore_axis_name="s"),
    scratch_types=[
        pltpu.VMEM_SHARED((NNP, HD), jnp.float32),
        pltpu.VMEM((CH, G), jnp.int32),
        pltpu.VMEM((CH, G), jnp.int32),
        pltpu.VMEM((CH, G), jnp.int32),
        pltpu.VMEM((CH, G), jnp.int32),
        pltpu.VMEM((CH, G), jnp.float32),
        pltpu.VMEM((CH, G), jnp.float32),
        pltpu.VMEM((CH, G), jnp.int32),
        pltpu.VMEM((CH, G), jnp.int32),
        pltpu.VMEM((CH, G, HD), jnp.float32),
        pltpu.VMEM((CH, G, HD), jnp.float32),
        pltpu.VMEM((ZR, HD), jnp.float32),
        pltpu.SemaphoreType.DMA,
        pltpu.SemaphoreType.DMA,
        pltpu.SemaphoreType.DMA,
        pltpu.SemaphoreType.DMA,
        pltpu.SemaphoreType.DMA,
        pltpu.SemaphoreType.DMA,
        pltpu.SemaphoreType.DMA,
    ],
    compiler_params=pltpu.CompilerParams(use_tc_tiling_on_sc=False),
)


def _dense_body(el_ref, eh_ref, gl_ref, gh_ref, wl_ref, wh_ref, on_ref,
                ol_ref, oh_ref):
    el = el_ref[...]
    eh = eh_ref[...]
    gl = gl_ref[...]
    gh = gh_ref[...]
    m = jnp.concatenate([el + gl, eh + gh, gl * el, gh * eh], axis=1)
    xl = jnp.dot(m, wl_ref[...])
    xh = jnp.dot(m, wh_ref[...])
    xl = jnp.where(xl >= 0, xl, 0.2 * xl)
    xh = jnp.where(xh >= 0, xh, 0.2 * xh)
    n2 = jnp.dot(xl * xl + xh * xh, on_ref[...])
    n = jnp.maximum(jnp.sqrt(n2), 1e-12)
    ol_ref[...] = xl / n
    oh_ref[...] = xh / n


def _dense(el, eh, gl, gh, wl, wh, ones_bd):
    blk = pl.BlockSpec((BR, 128), lambda i: (i, 0))
    wspec = pl.BlockSpec((512, 128), lambda i: (0, 0))
    ospec = pl.BlockSpec((128, 128), lambda i: (0, 0))
    hs = jax.ShapeDtypeStruct((ROWS128, 128), jnp.float32)
    return pl.pallas_call(
        _dense_body,
        grid=(ROWS128 // BR,),
        in_specs=[blk, blk, blk, blk, wspec, wspec, ospec],
        out_specs=(blk, blk),
        out_shape=(hs, hs),
    )(el, eh, gl, gh, wl, wh, ones_bd)


def _bgather_body(t0, t1, t2, t3, t4, t5, t6, t7, uid_hbm, iid_hbm,
                  u_out, i_out, uidv, iidv, rbuf, gsem):
    c = lax.axis_index("c")
    s = lax.axis_index("s")
    wid = s * NC + c
    base = wid * BPW
    pltpu.sync_copy(uid_hbm.at[pl.ds(base, BPW)], uidv)
    pltpu.sync_copy(iid_hbm.at[pl.ds(base, BPW)], iidv)
    off = jnp.full((16,), NN // 2, jnp.int32)
    for j in range(BPW // 16):
        iidv[pl.ds(j * 16, 16)] = iidv[pl.ds(j * 16, 16)] + off
    tbls = (t0, t1, t2, t3, t4, t5, t6, t7)
    for p in range(8):
        pltpu.async_copy(tbls[p].at[uidv], rbuf.at[p], gsem)
    for p in range(8):
        pltpu.make_async_copy(tbls[p].at[uidv], rbuf.at[p], gsem).wait()
        pltpu.sync_copy(rbuf.at[p],
                        u_out.at[pl.ds(base, BPW), pl.ds(p * HD, HD)])
    for p in range(8):
        pltpu.async_copy(tbls[p].at[iidv], rbuf.at[p], gsem)
    for p in range(8):
        pltpu.make_async_copy(tbls[p].at[iidv], rbuf.at[p], gsem).wait()
        pltpu.sync_copy(rbuf.at[p],
                        i_out.at[pl.ds(base, BPW), pl.ds(p * HD, HD)])


_bgather = pl.kernel(
    _bgather_body,
    out_type=(jax.ShapeDtypeStruct((BATCH, 128), jnp.float32),
              jax.ShapeDtypeStruct((BATCH, 128), jnp.float32)),
    mesh=plsc.VectorSubcoreMesh(core_axis_name="c", subcore_axis_name="s"),
    scratch_types=[
        pltpu.VMEM((BPW,), jnp.int32),
        pltpu.VMEM((BPW,), jnp.int32),
        pltpu.VMEM((8, BPW, HD), jnp.float32),
        pltpu.SemaphoreType.DMA,
    ],
    compiler_params=pltpu.CompilerParams(use_tc_tiling_on_sc=False),
)


def _dense_math(e, g, w1, w2):
    x = jnp.dot(e + g, w1) + jnp.dot(g * e, w2)
    x = jnp.where(x >= 0, x, 0.2 * x)
    n = jnp.maximum(jnp.sqrt(jnp.sum(x * x, axis=1, keepdims=True)), 1e-12)
    return x / n


def _score_body(u_ref, i_ref, w1_ref, w2_ref, o_ref):
    w1 = w1_ref[...]
    w2 = w2_ref[...]
    u = u_ref[...]
    i = i_ref[...]
    u3 = _dense_math(u[:, 64:96], u[:, 96:128], w1, w2)
    i3 = _dense_math(i[:, 64:96], i[:, 96:128], w1, w2)
    ub = jnp.concatenate([u[:, :96], u3], axis=1)
    ib = jnp.concatenate([i[:, :96], i3], axis=1)
    o_ref[...] = jnp.dot(ub, ib.T)


def _score(u_full, i_full, W1Tl, W2Tl):
    BU = 512
    wspec = pl.BlockSpec((D, D), lambda i, j: (0, 0))
    return pl.pallas_call(
        _score_body,
        grid=(BATCH // BU, BATCH // BU),
        in_specs=[pl.BlockSpec((BU, 128), lambda i, j: (i, 0)),
                  pl.BlockSpec((BU, 128), lambda i, j: (j, 0)),
                  wspec, wspec],
        out_specs=pl.BlockSpec((BU, BU), lambda i, j: (i, j)),
        out_shape=jax.ShapeDtypeStruct((BATCH, BATCH), jnp.float32),
    )(u_full, i_full, W1Tl, W2Tl)


def _bd(a):
    return jnp.kron(jnp.eye(8, dtype=a.dtype), a)


def kernel(user_ids, item_ids, user_embed, item_embed, W1, W2,
           edge_row, edge_col, edge_val):
    ne = edge_row.shape[0]
    pad = NE_PAD - ne
    row2 = jnp.pad(edge_row.astype(jnp.int32), (0, pad)).reshape(GROUPS, G)
    col2 = jnp.pad(edge_col.astype(jnp.int32), (0, pad)).reshape(GROUPS, G)
    val2 = jnp.pad(edge_val, (0, pad)).reshape(GROUPS, G)
    n_layers = W1.shape[0]
    W1T = jnp.swapaxes(W1, 1, 2)
    W2T = jnp.swapaxes(W2, 1, 2)
    wls, whs, = [], []
    for l in range(n_layers - 1):
        w1t, w2t = W1T[l], W2T[l]
        wls.append(jnp.concatenate(
            [_bd(w1t[:HD, :HD]), _bd(w1t[HD:, :HD]),
             _bd(w2t[:HD, :HD]), _bd(w2t[HD:, :HD])], axis=0))
        whs.append(jnp.concatenate(
            [_bd(w1t[:HD, HD:]), _bd(w1t[HD:, HD:]),
             _bd(w2t[:HD, HD:]), _bd(w2t[HD:, HD:])], axis=0))
    ones_bd = _bd(jnp.ones((HD, HD), jnp.float32))

    zpad = jnp.zeros((NNP - NN, HD), jnp.float32)
    el = jnp.concatenate([user_embed[:, :HD], item_embed[:, :HD], zpad])
    eh = jnp.concatenate([user_embed[:, HD:], item_embed[:, HD:], zpad])
    tbls = [el, eh]
    for l in range(n_layers - 1):
        gl, gh = _spmm(el, eh, row2, col2, val2)
        ol, oh = _dense(el.reshape(ROWS128, 128), eh.reshape(ROWS128, 128),
                        gl.reshape(ROWS128, 128), gh.reshape(ROWS128, 128),
                        wls[l], whs[l], ones_bd)
        el = ol.reshape(NNP, HD)
        eh = oh.reshape(NNP, HD)
        tbls += [el, eh]
    g3l, g3h = _spmm(el, eh, row2, col2, val2)
    tbls += [g3l, g3h]

    uid = user_ids.astype(jnp.int32)
    iid = item_ids.astype(jnp.int32)
    u_full, i_full = _bgather(*tbls, uid, iid)
    return _score(u_full, i_full, W1T[n_layers - 1], W2T[n_layers - 1])


# speedup vs baseline: 21.7595x; 1.1065x over previous
import jax
import jax.numpy as jnp
from jax import lax
from jax.experimental import pallas as pl
from jax.experimental.pallas import tpu as pltpu
from jax.experimental.pallas import tpu_sc as plsc

NN = 100000
NNP = 100352
D = 32
HD = 16
NS = 16
NC = 2
G = 128
CH = 4
CB = CH * G
NE = 1600000
TOTCH = NE // CB
BASECH = TOTCH // NS
EXTRA = TOTCH - BASECH * NS
PAIRS = (BASECH - 1) // 2
ZR = 128
ZSTRIPE = NNP // NS
ROWS128 = NNP * HD // 128
BR = ROWS128 // 8
BATCH = 4096
BPW = BATCH // (NC * NS)


def _spmm_body(el_hbm, eh_hbm, row_hbm, col_hbm, val_hbm, out_l, out_h,
               acc, colb0, colb1, rowb0, rowb1, valb0, valb1,
               idxb0, idxb1, rowsb0, rowsb1, zrow,
               sem_l0, sem_l1, sem_g0, sem_g1, sem_s0, sem_s1, sem_z):
    c = lax.axis_index("c")
    s = lax.axis_index("s")
    q0 = s * BASECH

    def load_chunk(q, colb, rowb, valb, sem):
        woff = q * CB
        pltpu.async_copy(row_hbm.at[pl.ds(woff, CB)], rowb, sem)
        pltpu.async_copy(col_hbm.at[pl.ds(woff, CB)], colb, sem)
        pltpu.async_copy(val_hbm.at[pl.ds(woff, CB)], valb, sem)

    def wait_chunk(colb, rowb, valb, sem):
        pltpu.make_async_copy(row_hbm.at[pl.ds(0, CB)], rowb, sem).wait()
        pltpu.make_async_copy(col_hbm.at[pl.ds(0, CB)], colb, sem).wait()
        pltpu.make_async_copy(val_hbm.at[pl.ds(0, CB)], valb, sem).wait()

    def fire(colb, rowsb, sem):
        @pl.when(c == 0)
        def _():
            for g2 in range(CH):
                pltpu.async_copy(el_hbm.at[colb.at[pl.ds(g2 * G, G)]],
                                 rowsb.at[g2], sem)

        @pl.when(c == 1)
        def _():
            for g2 in range(CH):
                pltpu.async_copy(eh_hbm.at[colb.at[pl.ds(g2 * G, G)]],
                                 rowsb.at[g2], sem)

    def drain(colb, rowsb, sem):
        for g2 in range(CH):
            pltpu.make_async_copy(el_hbm.at[colb.at[pl.ds(g2 * G, G)]],
                                  rowsb.at[g2], sem).wait()

    def scale(valb, rowsb):
        for g2 in range(CH):
            def sbody(j, carry, g2=g2):
                v16 = valb[pl.ds(g2 * G + j * 16, 16)]
                eb = j * 16
                for kk in range(16):
                    rowsb[g2, eb + kk] = rowsb[g2, eb + kk] * v16[kk]
                return carry

            lax.fori_loop(0, G // 16, sbody, 0)

    def copy_idx(rowb, idxb):
        for g2 in range(CH):
            def cbody(j, carry, g2=g2):
                idxb[g2, pl.ds(j * 16, 16)] = rowb[pl.ds(g2 * G + j * 16, 16)]
                return carry

            lax.fori_loop(0, G // 16, cbody, 0)

    def fire_scatter(rowsb, idxb, sem):
        for g2 in range(CH):
            pltpu.async_copy(rowsb.at[g2], acc.at[idxb.at[g2]], sem, add=True)

    def wait_scatter(rowsb, idxb, sem):
        for g2 in range(CH):
            pltpu.make_async_copy(rowsb.at[g2], acc.at[idxb.at[g2]],
                                  sem).wait()

    load_chunk(q0, colb0, rowb0, valb0, sem_l0)
    load_chunk(q0 + 1, colb1, rowb1, valb1, sem_l1)

    zeros16 = jnp.zeros((16,), jnp.float32)

    def zb(i, carry):
        zrow[i] = zeros16
        return carry

    lax.fori_loop(0, ZR, zb, 0)
    zbase = s * ZSTRIPE
    for k in range(ZSTRIPE // ZR):
        pltpu.async_copy(zrow, acc.at[pl.ds(zbase + k * ZR, ZR)], sem_z)
    wait_chunk(colb0, rowb0, valb0, sem_l0)
    for k in range(ZSTRIPE // ZR):
        pltpu.make_async_copy(zrow, acc.at[pl.ds(zbase, ZR)], sem_z).wait()
    plsc.subcore_barrier()
    fire(colb0, rowsb0, sem_g0)

    def body(k2, carry):
        a = 2 * k2
        b = a + 1

        @pl.when(k2 > 0)
        def _():
            wait_scatter(rowsb1, idxb1, sem_s1)

        wait_chunk(colb1, rowb1, valb1, sem_l1)
        fire(colb1, rowsb1, sem_g1)
        drain(colb0, rowsb0, sem_g0)
        scale(valb0, rowsb0)
        copy_idx(rowb0, idxb0)
        fire_scatter(rowsb0, idxb0, sem_s0)

        @pl.when(k2 < PAIRS - 1)
        def _():
            load_chunk(q0 + a + 2, colb0, rowb0, valb0, sem_l0)

        drain(colb1, rowsb1, sem_g1)
        scale(valb1, rowsb1)
        copy_idx(rowb1, idxb1)
        wait_scatter(rowsb0, idxb0, sem_s0)
        fire_scatter(rowsb1, idxb1, sem_s1)

        @pl.when(k2 < PAIRS - 1)
        def _():
            wait_chunk(colb0, rowb0, valb0, sem_l0)
            fire(colb0, rowsb0, sem_g0)
            load_chunk(q0 + b + 2, colb1, rowb1, valb1, sem_l1)

        return carry

    lax.fori_loop(0, PAIRS, body, 0)
    wait_scatter(rowsb1, idxb1, sem_s1)

    def do_chunk_sync(q):
        pltpu.sync_copy(row_hbm.at[pl.ds(q * CB, CB)], rowb0)
        pltpu.sync_copy(col_hbm.at[pl.ds(q * CB, CB)], colb0)
        pltpu.sync_copy(val_hbm.at[pl.ds(q * CB, CB)], valb0)
        fire(colb0, rowsb0, sem_g0)
        drain(colb0, rowsb0, sem_g0)
        scale(valb0, rowsb0)
        copy_idx(rowb0, idxb0)
        fire_scatter(rowsb0, idxb0, sem_s0)
        wait_scatter(rowsb0, idxb0, sem_s0)

    do_chunk_sync(q0 + BASECH - 1)

    @pl.when(s < EXTRA)
    def _():
        do_chunk_sync(NS * BASECH + s)

    plsc.subcore_barrier()

    ob = s * ZSTRIPE

    @pl.when(c == 0)
    def _():
        pltpu.sync_copy(acc.at[pl.ds(ob, ZSTRIPE)],
                        out_l.at[pl.ds(ob, ZSTRIPE)])

    @pl.when(c == 1)
    def _():
        pltpu.sync_copy(acc.at[pl.ds(ob, ZSTRIPE)],
                        out_h.at[pl.ds(ob, ZSTRIPE)])


_spmm = pl.kernel(
    _spmm_body,
    out_type=(jax.ShapeDtypeStruct((NNP, HD), jnp.float32),
              jax.ShapeDtypeStruct((NNP, HD), jnp.float32)),
    mesh=plsc.VectorSubcoreMesh(core_axis_name="c", subcore_axis_name="s"),
    scratch_types=[
        pltpu.VMEM_SHARED((NNP, HD), jnp.float32),
        pltpu.VMEM((CB,), jnp.int32),
        pltpu.VMEM((CB,), jnp.int32),
        pltpu.VMEM((CB,), jnp.int32),
        pltpu.VMEM((CB,), jnp.int32),
        pltpu.VMEM((CB,), jnp.float32),
        pltpu.VMEM((CB,), jnp.float32),
        pltpu.VMEM((CH, G), jnp.int32),
        pltpu.VMEM((CH, G), jnp.int32),
        pltpu.VMEM((CH, G, HD), jnp.float32),
        pltpu.VMEM((CH, G, HD), jnp.float32),
        pltpu.VMEM((ZR, HD), jnp.float32),
        pltpu.SemaphoreType.DMA,
        pltpu.SemaphoreType.DMA,
        pltpu.SemaphoreType.DMA,
        pltpu.SemaphoreType.DMA,
        pltpu.SemaphoreType.DMA,
        pltpu.SemaphoreType.DMA,
        pltpu.SemaphoreType.DMA,
    ],
    compiler_params=pltpu.CompilerParams(use_tc_tiling_on_sc=False),
)


def _dense_body(el_ref, eh_ref, gl_ref, gh_ref, wl_ref, wh_ref, on_ref,
                ol_ref, oh_ref):
    el = el_ref[...]
    eh = eh_ref[...]
    gl = gl_ref[...]
    gh = gh_ref[...]
    m = jnp.concatenate([el + gl, eh + gh, gl * el, gh * eh], axis=1)
    xl = jnp.dot(m, wl_ref[...])
    xh = jnp.dot(m, wh_ref[...])
    xl = jnp.where(xl >= 0, xl, 0.2 * xl)
    xh = jnp.where(xh >= 0, xh, 0.2 * xh)
    n2 = jnp.dot(xl * xl + xh * xh, on_ref[...])
    n = jnp.maximum(jnp.sqrt(n2), 1e-12)
    ol_ref[...] = xl / n
    oh_ref[...] = xh / n


def _dense(el, eh, gl, gh, wl, wh, ones_bd):
    blk = pl.BlockSpec((BR, 128), lambda i: (i, 0))
    wspec = pl.BlockSpec((512, 128), lambda i: (0, 0))
    ospec = pl.BlockSpec((128, 128), lambda i: (0, 0))
    hs = jax.ShapeDtypeStruct((ROWS128, 128), jnp.float32)
    return pl.pallas_call(
        _dense_body,
        grid=(ROWS128 // BR,),
        in_specs=[blk, blk, blk, blk, wspec, wspec, ospec],
        out_specs=(blk, blk),
        out_shape=(hs, hs),
    )(el, eh, gl, gh, wl, wh, ones_bd)


def _bgather_body(t0, t1, t2, t3, t4, t5, t6, t7, uid_hbm, iid_hbm,
                  u_out, i_out, uidv, iidv, rbuf, gsem):
    c = lax.axis_index("c")
    s = lax.axis_index("s")
    wid = s * NC + c
    base = wid * BPW
    pltpu.sync_copy(uid_hbm.at[pl.ds(base, BPW)], uidv)
    pltpu.sync_copy(iid_hbm.at[pl.ds(base, BPW)], iidv)
    off = jnp.full((16,), NN // 2, jnp.int32)
    for j in range(BPW // 16):
        iidv[pl.ds(j * 16, 16)] = iidv[pl.ds(j * 16, 16)] + off
    tbls = (t0, t1, t2, t3, t4, t5, t6, t7)
    for p in range(8):
        pltpu.async_copy(tbls[p].at[uidv], rbuf.at[p], gsem)
    for p in range(8):
        pltpu.make_async_copy(tbls[p].at[uidv], rbuf.at[p], gsem).wait()
        pltpu.sync_copy(rbuf.at[p],
                        u_out.at[pl.ds(base, BPW), pl.ds(p * HD, HD)])
    for p in range(8):
        pltpu.async_copy(tbls[p].at[iidv], rbuf.at[p], gsem)
    for p in range(8):
        pltpu.make_async_copy(tbls[p].at[iidv], rbuf.at[p], gsem).wait()
        pltpu.sync_copy(rbuf.at[p],
                        i_out.at[pl.ds(base, BPW), pl.ds(p * HD, HD)])


_bgather = pl.kernel(
    _bgather_body,
    out_type=(jax.ShapeDtypeStruct((BATCH, 128), jnp.float32),
              jax.ShapeDtypeStruct((BATCH, 128), jnp.float32)),
    mesh=plsc.VectorSubcoreMesh(core_axis_name="c", subcore_axis_name="s"),
    scratch_types=[
        pltpu.VMEM((BPW,), jnp.int32),
        pltpu.VMEM((BPW,), jnp.int32),
        pltpu.VMEM((8, BPW, HD), jnp.float32),
        pltpu.SemaphoreType.DMA,
    ],
    compiler_params=pltpu.CompilerParams(use_tc_tiling_on_sc=False),
)


def _dense_math(e, g, w1, w2):
    x = jnp.dot(e + g, w1) + jnp.dot(g * e, w2)
    x = jnp.where(x >= 0, x, 0.2 * x)
    n = jnp.maximum(jnp.sqrt(jnp.sum(x * x, axis=1, keepdims=True)), 1e-12)
    return x / n


def _fixup_body(u_ref, i_ref, w1_ref, w2_ref, uo_ref, io_ref):
    w1 = w1_ref[...]
    w2 = w2_ref[...]
    u = u_ref[...]
    i = i_ref[...]
    u3 = _dense_math(u[:, 64:96], u[:, 96:128], w1, w2)
    i3 = _dense_math(i[:, 64:96], i[:, 96:128], w1, w2)
    uo_ref[...] = jnp.concatenate([u[:, :96], u3], axis=1)
    io_ref[...] = jnp.concatenate([i[:, :96], i3], axis=1)


def _fixup(u_full, i_full, W1Tl, W2Tl):
    BF = 512
    blk = pl.BlockSpec((BF, 128), lambda i: (i, 0))
    wspec = pl.BlockSpec((D, D), lambda i: (0, 0))
    fs = jax.ShapeDtypeStruct((BATCH, 128), jnp.float32)
    return pl.pallas_call(
        _fixup_body,
        grid=(BATCH // BF,),
        in_specs=[blk, blk, wspec, wspec],
        out_specs=(blk, blk),
        out_shape=(fs, fs),
    )(u_full, i_full, W1Tl, W2Tl)


def _score_body(u_ref, i_ref, o_ref):
    o_ref[...] = jnp.dot(u_ref[...], i_ref[...].T)


def _score(u_full, i_full):
    BU = 512
    return pl.pallas_call(
        _score_body,
        grid=(BATCH // BU, BATCH // BU),
        in_specs=[pl.BlockSpec((BU, 128), lambda i, j: (i, 0)),
                  pl.BlockSpec((BU, 128), lambda i, j: (j, 0))],
        out_specs=pl.BlockSpec((BU, BU), lambda i, j: (i, j)),
        out_shape=jax.ShapeDtypeStruct((BATCH, BATCH), jnp.float32),
    )(u_full, i_full)


def _bd(a):
    return jnp.kron(jnp.eye(8, dtype=a.dtype), a)


def kernel(user_ids, item_ids, user_embed, item_embed, W1, W2,
           edge_row, edge_col, edge_val):
    row2 = edge_row.astype(jnp.int32)
    col2 = edge_col.astype(jnp.int32)
    val2 = edge_val
    n_layers = W1.shape[0]
    W1T = jnp.swapaxes(W1, 1, 2)
    W2T = jnp.swapaxes(W2, 1, 2)
    wls, whs, = [], []
    for l in range(n_layers - 1):
        w1t, w2t = W1T[l], W2T[l]
        wls.append(jnp.concatenate(
            [_bd(w1t[:HD, :HD]), _bd(w1t[HD:, :HD]),
             _bd(w2t[:HD, :HD]), _bd(w2t[HD:, :HD])], axis=0))
        whs.append(jnp.concatenate(
            [_bd(w1t[:HD, HD:]), _bd(w1t[HD:, HD:]),
             _bd(w2t[:HD, HD:]), _bd(w2t[HD:, HD:])], axis=0))
    ones_bd = _bd(jnp.ones((HD, HD), jnp.float32))

    zpad = jnp.zeros((NNP - NN, HD), jnp.float32)
    el = jnp.concatenate([user_embed[:, :HD], item_embed[:, :HD], zpad])
    eh = jnp.concatenate([user_embed[:, HD:], item_embed[:, HD:], zpad])
    tbls = [el, eh]
    for l in range(n_layers - 1):
        gl, gh = _spmm(el, eh, row2, col2, val2)
        ol, oh = _dense(el.reshape(ROWS128, 128), eh.reshape(ROWS128, 128),
                        gl.reshape(ROWS128, 128), gh.reshape(ROWS128, 128),
                        wls[l], whs[l], ones_bd)
        el = ol.reshape(NNP, HD)
        eh = oh.reshape(NNP, HD)
        tbls += [el, eh]
    g3l, g3h = _spmm(el, eh, row2, col2, val2)
    tbls += [g3l, g3h]

    uid = user_ids.astype(jnp.int32)
    iid = item_ids.astype(jnp.int32)
    u_full, i_full = _bgather(*tbls, uid, iid)
    u_fix, i_fix = _fixup(u_full, i_full, W1T[n_layers - 1],
                          W2T[n_layers - 1])
    return _score(u_fix, i_fix)


# speedup vs baseline: 22.6558x; 1.0412x over previous
import jax
import jax.numpy as jnp
from jax import lax
from jax.experimental import pallas as pl
from jax.experimental.pallas import tpu as pltpu
from jax.experimental.pallas import tpu_sc as plsc

NN = 100000
NNP = 100352
D = 32
HD = 16
NS = 16
NC = 2
G = 128
CH = 4
CB = CH * G
NE = 1600000
TOTCH = NE // CB
BASECH = TOTCH // NS
EXTRA = TOTCH - BASECH * NS
PAIRS = (BASECH - 1) // 2
ZR = 128
ZSTRIPE = NNP // NS
ROWS128 = NNP * HD // 128
BR = ROWS128 // 8
BATCH = 4096
BPW = BATCH // (NC * NS)


def _spmm_body(el_hbm, eh_hbm, row_hbm, col_hbm, val_hbm, out_l, out_h,
               acc, colb0, colb1, rowb0, rowb1, valb0, valb1,
               idxb0, idxb1, rowsb0, rowsb1, zrow,
               sem_l0, sem_l1, sem_g0, sem_g1, sem_s0, sem_s1, sem_z):
    c = lax.axis_index("c")
    s = lax.axis_index("s")
    q0 = s * BASECH

    def load_chunk(q, colb, rowb, valb, sem):
        woff = q * CB
        pltpu.async_copy(row_hbm.at[pl.ds(woff, CB)], rowb, sem)
        pltpu.async_copy(col_hbm.at[pl.ds(woff, CB)], colb, sem)
        pltpu.async_copy(val_hbm.at[pl.ds(woff, CB)], valb, sem)

    def wait_chunk(colb, rowb, valb, sem):
        pltpu.make_async_copy(row_hbm.at[pl.ds(0, CB)], rowb, sem).wait()
        pltpu.make_async_copy(col_hbm.at[pl.ds(0, CB)], colb, sem).wait()
        pltpu.make_async_copy(val_hbm.at[pl.ds(0, CB)], valb, sem).wait()

    def fire(colb, rowsb, sem):
        @pl.when(c == 0)
        def _():
            for g2 in range(CH):
                pltpu.async_copy(el_hbm.at[colb.at[pl.ds(g2 * G, G)]],
                                 rowsb.at[g2], sem)

        @pl.when(c == 1)
        def _():
            for g2 in range(CH):
                pltpu.async_copy(eh_hbm.at[colb.at[pl.ds(g2 * G, G)]],
                                 rowsb.at[g2], sem)

    def drain(colb, rowsb, sem):
        for g2 in range(CH):
            pltpu.make_async_copy(el_hbm.at[colb.at[pl.ds(g2 * G, G)]],
                                  rowsb.at[g2], sem).wait()

    def scale(valb, rowsb):
        for g2 in range(CH):
            def sbody(j, carry, g2=g2):
                v16 = valb[pl.ds(g2 * G + j * 16, 16)]
                eb = j * 16
                for kk in range(16):
                    rowsb[g2, eb + kk] = rowsb[g2, eb + kk] * v16[kk]
                return carry

            lax.fori_loop(0, G // 16, sbody, 0)

    def copy_idx(rowb, idxb):
        for g2 in range(CH):
            def cbody(j, carry, g2=g2):
                idxb[g2, pl.ds(j * 16, 16)] = rowb[pl.ds(g2 * G + j * 16, 16)]
                return carry

            lax.fori_loop(0, G // 16, cbody, 0)

    def fire_scatter(rowsb, idxb, sem):
        for g2 in range(CH):
            pltpu.async_copy(rowsb.at[g2], acc.at[idxb.at[g2]], sem, add=True)

    def wait_scatter(rowsb, idxb, sem):
        for g2 in range(CH):
            pltpu.make_async_copy(rowsb.at[g2], acc.at[idxb.at[g2]],
                                  sem).wait()

    load_chunk(q0, colb0, rowb0, valb0, sem_l0)
    load_chunk(q0 + 1, colb1, rowb1, valb1, sem_l1)

    zeros16 = jnp.zeros((16,), jnp.float32)

    def zb(i, carry):
        zrow[i] = zeros16
        return carry

    lax.fori_loop(0, ZR, zb, 0)
    zbase = s * ZSTRIPE
    for k in range(ZSTRIPE // ZR):
        pltpu.async_copy(zrow, acc.at[pl.ds(zbase + k * ZR, ZR)], sem_z)
    wait_chunk(colb0, rowb0, valb0, sem_l0)
    for k in range(ZSTRIPE // ZR):
        pltpu.make_async_copy(zrow, acc.at[pl.ds(zbase, ZR)], sem_z).wait()
    plsc.subcore_barrier()
    fire(colb0, rowsb0, sem_g0)

    def body(k2, carry):
        a = 2 * k2
        b = a + 1

        @pl.when(k2 > 0)
        def _():
            wait_scatter(rowsb1, idxb1, sem_s1)

        wait_chunk(colb1, rowb1, valb1, sem_l1)
        fire(colb1, rowsb1, sem_g1)
        drain(colb0, rowsb0, sem_g0)
        scale(valb0, rowsb0)
        copy_idx(rowb0, idxb0)
        fire_scatter(rowsb0, idxb0, sem_s0)

        @pl.when(k2 < PAIRS - 1)
        def _():
            load_chunk(q0 + a + 2, colb0, rowb0, valb0, sem_l0)

        drain(colb1, rowsb1, sem_g1)
        scale(valb1, rowsb1)
        copy_idx(rowb1, idxb1)
        wait_scatter(rowsb0, idxb0, sem_s0)
        fire_scatter(rowsb1, idxb1, sem_s1)

        @pl.when(k2 < PAIRS - 1)
        def _():
            wait_chunk(colb0, rowb0, valb0, sem_l0)
            fire(colb0, rowsb0, sem_g0)
            load_chunk(q0 + b + 2, colb1, rowb1, valb1, sem_l1)

        return carry

    lax.fori_loop(0, PAIRS, body, 0)
    wait_scatter(rowsb1, idxb1, sem_s1)

    def do_chunk_sync(q):
        pltpu.sync_copy(row_hbm.at[pl.ds(q * CB, CB)], rowb0)
        pltpu.sync_copy(col_hbm.at[pl.ds(q * CB, CB)], colb0)
        pltpu.sync_copy(val_hbm.at[pl.ds(q * CB, CB)], valb0)
        fire(colb0, rowsb0, sem_g0)
        drain(colb0, rowsb0, sem_g0)
        scale(valb0, rowsb0)
        copy_idx(rowb0, idxb0)
        fire_scatter(rowsb0, idxb0, sem_s0)
        wait_scatter(rowsb0, idxb0, sem_s0)

    do_chunk_sync(q0 + BASECH - 1)

    @pl.when(s < EXTRA)
    def _():
        do_chunk_sync(NS * BASECH + s)

    plsc.subcore_barrier()

    ob = s * ZSTRIPE

    @pl.when(c == 0)
    def _():
        pltpu.sync_copy(acc.at[pl.ds(ob, ZSTRIPE)],
                        out_l.at[pl.ds(ob, ZSTRIPE)])

    @pl.when(c == 1)
    def _():
        pltpu.sync_copy(acc.at[pl.ds(ob, ZSTRIPE)],
                        out_h.at[pl.ds(ob, ZSTRIPE)])


_spmm = pl.kernel(
    _spmm_body,
    out_type=(jax.ShapeDtypeStruct((NNP, HD), jnp.float32),
              jax.ShapeDtypeStruct((NNP, HD), jnp.float32)),
    mesh=plsc.VectorSubcoreMesh(core_axis_name="c", subcore_axis_name="s"),
    scratch_types=[
        pltpu.VMEM_SHARED((NNP, HD), jnp.float32),
        pltpu.VMEM((CB,), jnp.int32),
        pltpu.VMEM((CB,), jnp.int32),
        pltpu.VMEM((CB,), jnp.int32),
        pltpu.VMEM((CB,), jnp.int32),
        pltpu.VMEM((CB,), jnp.float32),
        pltpu.VMEM((CB,), jnp.float32),
        pltpu.VMEM((CH, G), jnp.int32),
        pltpu.VMEM((CH, G), jnp.int32),
        pltpu.VMEM((CH, G, HD), jnp.float32),
        pltpu.VMEM((CH, G, HD), jnp.float32),
        pltpu.VMEM((ZR, HD), jnp.float32),
        pltpu.SemaphoreType.DMA,
        pltpu.SemaphoreType.DMA,
        pltpu.SemaphoreType.DMA,
        pltpu.SemaphoreType.DMA,
        pltpu.SemaphoreType.DMA,
        pltpu.SemaphoreType.DMA,
        pltpu.SemaphoreType.DMA,
    ],
    compiler_params=pltpu.CompilerParams(use_tc_tiling_on_sc=False),
)


SPLIT_B0 = 782
SPLIT_B1 = 781


def _split_body(userf, itemf, el3, eh3, chunk, zbuf):
    c = lax.axis_index("c")
    s = lax.axis_index("s")
    wid = s * NC + c
    w16 = wid % 16

    def do_split(table, ro, rb, B):
        pltpu.sync_copy(table.at[pl.ds(rb, B)], chunk.at[pl.ds(0, B)])
        for k in range(4):
            pltpu.sync_copy(chunk.at[pl.ds(0, B), pl.ds(k * 32, 16)],
                            el3.at[pl.ds(ro + rb, B), k])
            pltpu.sync_copy(chunk.at[pl.ds(0, B), pl.ds(k * 32 + 16, 16)],
                            eh3.at[pl.ds(ro + rb, B), k])

    @pl.when(wid < 16)
    def _():
        @pl.when(w16 < 4)
        def _():
            do_split(userf, 0, w16 * SPLIT_B0, SPLIT_B0)

        @pl.when(w16 >= 4)
        def _():
            do_split(userf, 0, 4 * SPLIT_B0 + (w16 - 4) * SPLIT_B1, SPLIT_B1)

    @pl.when(wid >= 16)
    def _():
        @pl.when(w16 < 4)
        def _():
            do_split(itemf, 12500, w16 * SPLIT_B0, SPLIT_B0)

        @pl.when(w16 >= 4)
        def _():
            do_split(itemf, 12500, 4 * SPLIT_B0 + (w16 - 4) * SPLIT_B1,
                     SPLIT_B1)

    @pl.when(wid == 31)
    def _():
        zeros16 = jnp.zeros((16,), jnp.float32)

        def zb(i, carry):
            for k in range(4):
                zbuf[i, k] = zeros16
            return carry

        lax.fori_loop(0, 88, zb, 0)
        pltpu.sync_copy(zbuf, el3.at[pl.ds(25000, 88)])
        pltpu.sync_copy(zbuf, eh3.at[pl.ds(25000, 88)])


_split = pl.kernel(
    _split_body,
    out_type=(jax.ShapeDtypeStruct((25088, 4, HD), jnp.float32),
              jax.ShapeDtypeStruct((25088, 4, HD), jnp.float32)),
    mesh=plsc.VectorSubcoreMesh(core_axis_name="c", subcore_axis_name="s"),
    scratch_types=[
        pltpu.VMEM((SPLIT_B0, 128), jnp.float32),
        pltpu.VMEM((88, 4, HD), jnp.float32),
    ],
    compiler_params=pltpu.CompilerParams(use_tc_tiling_on_sc=False),
)


def _dense_body(el_ref, eh_ref, gl_ref, gh_ref, wl_ref, wh_ref, on_ref,
                ol_ref, oh_ref):
    el = el_ref[...]
    eh = eh_ref[...]
    gl = gl_ref[...]
    gh = gh_ref[...]
    m = jnp.concatenate([el + gl, eh + gh, gl * el, gh * eh], axis=1)
    xl = jnp.dot(m, wl_ref[...])
    xh = jnp.dot(m, wh_ref[...])
    xl = jnp.where(xl >= 0, xl, 0.2 * xl)
    xh = jnp.where(xh >= 0, xh, 0.2 * xh)
    n2 = jnp.dot(xl * xl + xh * xh, on_ref[...])
    n = jnp.maximum(jnp.sqrt(n2), 1e-12)
    ol_ref[...] = xl / n
    oh_ref[...] = xh / n


def _dense(el, eh, gl, gh, wl, wh, ones_bd):
    blk = pl.BlockSpec((BR, 128), lambda i: (i, 0))
    wspec = pl.BlockSpec((512, 128), lambda i: (0, 0))
    ospec = pl.BlockSpec((128, 128), lambda i: (0, 0))
    hs = jax.ShapeDtypeStruct((ROWS128, 128), jnp.float32)
    return pl.pallas_call(
        _dense_body,
        grid=(ROWS128 // BR,),
        in_specs=[blk, blk, blk, blk, wspec, wspec, ospec],
        out_specs=(blk, blk),
        out_shape=(hs, hs),
    )(el, eh, gl, gh, wl, wh, ones_bd)


def _bgather_body(t0, t1, t2, t3, t4, t5, t6, t7, uid_hbm, iid_hbm,
                  u_out, i_out, uidv, iidv, rbuf, gsem):
    c = lax.axis_index("c")
    s = lax.axis_index("s")
    wid = s * NC + c
    base = wid * BPW
    pltpu.sync_copy(uid_hbm.at[pl.ds(base, BPW)], uidv)
    pltpu.sync_copy(iid_hbm.at[pl.ds(base, BPW)], iidv)
    off = jnp.full((16,), NN // 2, jnp.int32)
    for j in range(BPW // 16):
        iidv[pl.ds(j * 16, 16)] = iidv[pl.ds(j * 16, 16)] + off
    tbls = (t0, t1, t2, t3, t4, t5, t6, t7)
    for p in range(8):
        pltpu.async_copy(tbls[p].at[uidv], rbuf.at[p], gsem)
    for p in range(8):
        pltpu.make_async_copy(tbls[p].at[uidv], rbuf.at[p], gsem).wait()
        pltpu.sync_copy(rbuf.at[p],
                        u_out.at[pl.ds(base, BPW), pl.ds(p * HD, HD)])
    for p in range(8):
        pltpu.async_copy(tbls[p].at[iidv], rbuf.at[p], gsem)
    for p in range(8):
        pltpu.make_async_copy(tbls[p].at[iidv], rbuf.at[p], gsem).wait()
        pltpu.sync_copy(rbuf.at[p],
                        i_out.at[pl.ds(base, BPW), pl.ds(p * HD, HD)])


_bgather = pl.kernel(
    _bgather_body,
    out_type=(jax.ShapeDtypeStruct((BATCH, 128), jnp.float32),
              jax.ShapeDtypeStruct((BATCH, 128), jnp.float32)),
    mesh=plsc.VectorSubcoreMesh(core_axis_name="c", subcore_axis_name="s"),
    scratch_types=[
        pltpu.VMEM((BPW,), jnp.int32),
        pltpu.VMEM((BPW,), jnp.int32),
        pltpu.VMEM((8, BPW, HD), jnp.float32),
        pltpu.SemaphoreType.DMA,
    ],
    compiler_params=pltpu.CompilerParams(use_tc_tiling_on_sc=False),
)


def _dense_math(e, g, w1, w2):
    x = jnp.dot(e + g, w1) + jnp.dot(g * e, w2)
    x = jnp.where(x >= 0, x, 0.2 * x)
    n = jnp.maximum(jnp.sqrt(jnp.sum(x * x, axis=1, keepdims=True)), 1e-12)
    return x / n


def _fixup_body(u_ref, i_ref, w1_ref, w2_ref, uo_ref, io_ref):
    w1 = w1_ref[...]
    w2 = w2_ref[...]
    u = u_ref[...]
    i = i_ref[...]
    u3 = _dense_math(u[:, 64:96], u[:, 96:128], w1, w2)
    i3 = _dense_math(i[:, 64:96], i[:, 96:128], w1, w2)
    uo_ref[...] = jnp.concatenate([u[:, :96], u3], axis=1)
    io_ref[...] = jnp.concatenate([i[:, :96], i3], axis=1)


def _fixup(u_full, i_full, W1Tl, W2Tl):
    BF = 512
    blk = pl.BlockSpec((BF, 128), lambda i: (i, 0))
    wspec = pl.BlockSpec((D, D), lambda i: (0, 0))
    fs = jax.ShapeDtypeStruct((BATCH, 128), jnp.float32)
    return pl.pallas_call(
        _fixup_body,
        grid=(BATCH // BF,),
        in_specs=[blk, blk, wspec, wspec],
        out_specs=(blk, blk),
        out_shape=(fs, fs),
    )(u_full, i_full, W1Tl, W2Tl)


def _score_body(u_ref, i_ref, o_ref):
    o_ref[...] = jnp.dot(u_ref[...], i_ref[...].T)


def _score(u_full, i_full):
    BU = 512
    return pl.pallas_call(
        _score_body,
        grid=(BATCH // BU, BATCH // BU),
        in_specs=[pl.BlockSpec((BU, 128), lambda i, j: (i, 0)),
                  pl.BlockSpec((BU, 128), lambda i, j: (j, 0))],
        out_specs=pl.BlockSpec((BU, BU), lambda i, j: (i, j)),
        out_shape=jax.ShapeDtypeStruct((BATCH, BATCH), jnp.float32),
    )(u_full, i_full)


def _bd(a):
    return jnp.kron(jnp.eye(8, dtype=a.dtype), a)


def kernel(user_ids, item_ids, user_embed, item_embed, W1, W2,
           edge_row, edge_col, edge_val):
    row2 = edge_row.astype(jnp.int32)
    col2 = edge_col.astype(jnp.int32)
    val2 = edge_val
    n_layers = W1.shape[0]
    W1T = jnp.swapaxes(W1, 1, 2)
    W2T = jnp.swapaxes(W2, 1, 2)
    wls, whs, = [], []
    for l in range(n_layers - 1):
        w1t, w2t = W1T[l], W2T[l]
        wls.append(jnp.concatenate(
            [_bd(w1t[:HD, :HD]), _bd(w1t[HD:, :HD]),
             _bd(w2t[:HD, :HD]), _bd(w2t[HD:, :HD])], axis=0))
        whs.append(jnp.concatenate(
            [_bd(w1t[:HD, HD:]), _bd(w1t[HD:, HD:]),
             _bd(w2t[:HD, HD:]), _bd(w2t[HD:, HD:])], axis=0))
    ones_bd = _bd(jnp.ones((HD, HD), jnp.float32))

    el3, eh3 = _split(user_embed.reshape(12500, 128),
                      item_embed.reshape(12500, 128))
    el = el3.reshape(NNP, HD)
    eh = eh3.reshape(NNP, HD)
    tbls = [el, eh]
    for l in range(n_layers - 1):
        gl, gh = _spmm(el, eh, row2, col2, val2)
        ol, oh = _dense(el.reshape(ROWS128, 128), eh.reshape(ROWS128, 128),
                        gl.reshape(ROWS128, 128), gh.reshape(ROWS128, 128),
                        wls[l], whs[l], ones_bd)
        el = ol.reshape(NNP, HD)
        eh = oh.reshape(NNP, HD)
        tbls += [el, eh]
    g3l, g3h = _spmm(el, eh, row2, col2, val2)
    tbls += [g3l, g3h]

    uid = user_ids.astype(jnp.int32)
    iid = item_ids.astype(jnp.int32)
    u_full, i_full = _bgather(*tbls, uid, iid)
    u_fix, i_fix = _fixup(u_full, i_full, W1T[n_layers - 1],
                          W2T[n_layers - 1])
    return _score(u_fix, i_fix)
